# baseline (speedup 1.0000x reference)
import functools

import jax
import jax.numpy as jnp
from jax.experimental import pallas as pl
from jax.experimental.pallas import tpu as pltpu


_VMEM_LIMIT = 64 * 1024 * 1024
_BN_EPS = 1e-5
_CPAD = 128
_MPAD = 4


def _shift_lanes(a, d):
    if d == 0:
        return a
    C = a.shape[0]
    if d > 0:
        return jnp.concatenate(
            [a[:, d:], jnp.zeros((C, d), a.dtype)], axis=1)
    return jnp.concatenate(
        [jnp.zeros((C, -d), a.dtype), a[:, :a.shape[1] + d]], axis=1)


def _block_kernel(x_ref, wc_ref, b1_ref, pm_ref, w3cat_ref, b3p_ref,
                  o_ref, *, ksize, pad, W, Bb, inp):
    HW = x_ref.shape[-1]
    wc = wc_ref[...]
    b1 = b1_ref[...]
    w3cat = w3cat_ref[...]
    b3p = b3p_ref[...]

    zrows = jnp.zeros((_CPAD - inp, HW), jnp.bfloat16)
    for b in range(Bb):
        xb = x_ref[b].astype(jnp.bfloat16)
        r = jnp.dot(wc, xb, preferred_element_type=jnp.float32)
        proj = r[:inp]
        h = jnp.maximum(r[inp + _MPAD:] + b1, 0.0)
        hb0 = h.astype(jnp.bfloat16)
        hb = {ox: hb0 * pm_ref[i] for i, ox in
              enumerate(o for o in range(-pad, pad + 1) if o != 0)}
        hb[0] = hb0
        pieces = []
        for dy in range(ksize):
            oy = dy - pad
            for dx in range(ksize):
                ox = dx - pad
                pieces.append(_shift_lanes(hb[ox], oy * W + ox))
                pieces.append(zrows)
        stack = jnp.concatenate(pieces, axis=0)
        y = jnp.maximum(
            jnp.dot(w3cat, stack, preferred_element_type=jnp.float32) + b3p,
            0.0)
        o_ref[b, 2 * inp - y.shape[0]:] = y
        o_ref[b, :inp] = proj


def _const_spec(a):
    zeros = (0,) * a.ndim
    return pl.BlockSpec(a.shape, lambda b: zeros)


def _pick_block_batch(B, target_steps=32):
    cap = max(1, B // target_steps)
    for bb in range(cap, 0, -1):
        if B % bb == 0:
            return bb
    return 1


def _fold_bn(gamma, beta, mean, var, eps=_BN_EPS):
    scale = gamma / jnp.sqrt(var + eps)
    bias = beta - mean * scale
    return scale, bias


def kernel(x, main_w1, main_bn1_gamma, main_bn1_beta, main_bn1_mean,
           main_bn1_var, main_dw, main_bn2_gamma, main_bn2_beta, main_bn2_mean,
           main_bn2_var, main_w3, main_bn3_gamma, main_bn3_beta, main_bn3_mean,
           main_bn3_var):
    B, C, H, W = x.shape
    inp = C // 2
    mid = main_w1.shape[0]
    HW = H * W
    ksize = main_dw.shape[-1]
    pad = ksize // 2
    outputs = main_w3.shape[0]
    assert outputs == inp

    s1, b1 = _fold_bn(main_bn1_gamma, main_bn1_beta, main_bn1_mean, main_bn1_var)
    s2, b2 = _fold_bn(main_bn2_gamma, main_bn2_beta, main_bn2_mean, main_bn2_var)
    s3, b3 = _fold_bn(main_bn3_gamma, main_bn3_beta, main_bn3_mean, main_bn3_var)
    w1 = main_w1[:, :, 0, 0] * s1[:, None]
    dwf = main_dw[:, 0].reshape(mid, -1) * s2[:, None]
    w3f = main_w3[:, :, 0, 0] * s3[:, None]
    b3f = b3 + s3 * (main_w3[:, :, 0, 0] @ b2)

    w1e = jnp.stack([jnp.zeros_like(w1), w1], axis=-1) \
        .reshape(mid, C).astype(jnp.bfloat16)
    sel = jnp.stack([jnp.eye(inp, dtype=jnp.bfloat16),
                     jnp.zeros((inp, inp), jnp.bfloat16)], axis=-1) \
        .reshape(inp, C)
    wcomb = jnp.concatenate(
        [sel, jnp.zeros((_MPAD, C), jnp.bfloat16), w1e], axis=0)

    xpos = jnp.arange(W)
    pmrows = [jnp.tile(((xpos >= max(0, ox)) & (xpos < W + min(0, ox))), H)
              for ox in range(-pad, pad + 1) if ox != 0]
    pmask = jnp.stack(pmrows).astype(jnp.bfloat16)
    pmask = jnp.broadcast_to(pmask[:, None, :], (pmask.shape[0], mid, HW))

    kk = ksize * ksize
    core = w3f[:, None, :] * dwf.T[None, :, :]
    w3cat = jnp.pad(core, ((_MPAD, 0), (0, 0), (0, _CPAD - mid))) \
        .reshape(outputs + _MPAD, kk * _CPAD).astype(jnp.bfloat16)
    b3p = jnp.pad(b3f, (_MPAD, 0))[:, None]

    b1c = b1[:, None]
    Bb = _pick_block_batch(B)
    x3 = x.reshape(B, C, HW)

    params = (wcomb, b1c, pmask, w3cat, b3p)
    kern = functools.partial(_block_kernel, ksize=ksize, pad=pad, W=W, Bb=Bb,
                             inp=inp)
    out = pl.pallas_call(
        kern,
        out_shape=jax.ShapeDtypeStruct((B, C, HW), jnp.float32),
        grid=(B // Bb,),
        in_specs=[pl.BlockSpec((Bb, C, HW), lambda b: (b, 0, 0))]
                 + [_const_spec(a) for a in params],
        out_specs=pl.BlockSpec((Bb, C, HW), lambda b: (b, 0, 0)),
        compiler_params=pltpu.CompilerParams(
            dimension_semantics=("parallel",),
            vmem_limit_bytes=_VMEM_LIMIT),
    )(x3, *params)
    return out.reshape(B, C, H, W)


# speedup vs baseline: 1.0290x; 1.0290x over previous
import functools

import jax
import jax.numpy as jnp
from jax.experimental import pallas as pl
from jax.experimental.pallas import tpu as pltpu


_VMEM_LIMIT = 64 * 1024 * 1024
_BN_EPS = 1e-5
_CPAD = 128
_MPAD = 4


def _shift_lanes(a, d):
    if d == 0:
        return a
    C = a.shape[0]
    if d > 0:
        return jnp.concatenate(
            [a[:, d:], jnp.zeros((C, d), a.dtype)], axis=1)
    return jnp.concatenate(
        [jnp.zeros((C, -d), a.dtype), a[:, :a.shape[1] + d]], axis=1)


def _block_kernel(x_ref, wc_ref, b1_ref, pm_ref, w3cat_ref, b3p_ref,
                  o_ref, *, ksize, pad, W, Bb, inp):
    HW = x_ref.shape[-1]
    wc = wc_ref[...]
    b1 = b1_ref[...]
    w3cat = w3cat_ref[...]
    b3p = b3p_ref[...]

    zrows = jnp.zeros((_CPAD - inp, HW), jnp.bfloat16)
    for b in range(Bb):
        xb = x_ref[b].astype(jnp.bfloat16)
        r = jnp.dot(wc, xb, preferred_element_type=jnp.float32)
        proj = r[:inp]
        h = jnp.maximum(r[inp + _MPAD:] + b1, 0.0)
        hb0 = h.astype(jnp.bfloat16)
        hb = {ox: hb0 * pm_ref[i] for i, ox in
              enumerate(o for o in range(-pad, pad + 1) if o != 0)}
        hb[0] = hb0
        pieces = []
        for dy in range(ksize):
            oy = dy - pad
            for dx in range(ksize):
                ox = dx - pad
                pieces.append(_shift_lanes(hb[ox], oy * W + ox))
                pieces.append(zrows)
        stack = jnp.concatenate(pieces, axis=0)
        y = jnp.maximum(
            jnp.dot(w3cat, stack, preferred_element_type=jnp.float32) + b3p,
            0.0)
        o_ref[b, 2 * inp - y.shape[0]:] = y
        o_ref[b, :inp] = proj


def _const_spec(a):
    zeros = (0,) * a.ndim
    return pl.BlockSpec(a.shape, lambda b: zeros)


def _pick_block_batch(B, target_steps=8):
    cap = max(1, B // target_steps)
    for bb in range(cap, 0, -1):
        if B % bb == 0:
            return bb
    return 1


def _fold_bn(gamma, beta, mean, var, eps=_BN_EPS):
    scale = gamma / jnp.sqrt(var + eps)
    bias = beta - mean * scale
    return scale, bias


def kernel(x, main_w1, main_bn1_gamma, main_bn1_beta, main_bn1_mean,
           main_bn1_var, main_dw, main_bn2_gamma, main_bn2_beta, main_bn2_mean,
           main_bn2_var, main_w3, main_bn3_gamma, main_bn3_beta, main_bn3_mean,
           main_bn3_var):
    B, C, H, W = x.shape
    inp = C // 2
    mid = main_w1.shape[0]
    HW = H * W
    ksize = main_dw.shape[-1]
    pad = ksize // 2
    outputs = main_w3.shape[0]
    assert outputs == inp

    s1, b1 = _fold_bn(main_bn1_gamma, main_bn1_beta, main_bn1_mean, main_bn1_var)
    s2, b2 = _fold_bn(main_bn2_gamma, main_bn2_beta, main_bn2_mean, main_bn2_var)
    s3, b3 = _fold_bn(main_bn3_gamma, main_bn3_beta, main_bn3_mean, main_bn3_var)
    w1 = main_w1[:, :, 0, 0] * s1[:, None]
    dwf = main_dw[:, 0].reshape(mid, -1) * s2[:, None]
    w3f = main_w3[:, :, 0, 0] * s3[:, None]
    b3f = b3 + s3 * (main_w3[:, :, 0, 0] @ b2)

    w1e = jnp.stack([jnp.zeros_like(w1), w1], axis=-1) \
        .reshape(mid, C).astype(jnp.bfloat16)
    sel = jnp.stack([jnp.eye(inp, dtype=jnp.bfloat16),
                     jnp.zeros((inp, inp), jnp.bfloat16)], axis=-1) \
        .reshape(inp, C)
    wcomb = jnp.concatenate(
        [sel, jnp.zeros((_MPAD, C), jnp.bfloat16), w1e], axis=0)

    xpos = jnp.arange(W)
    pmrows = [jnp.tile(((xpos >= max(0, ox)) & (xpos < W + min(0, ox))), H)
              for ox in range(-pad, pad + 1) if ox != 0]
    pmask = jnp.stack(pmrows).astype(jnp.bfloat16)
    pmask = jnp.broadcast_to(pmask[:, None, :], (pmask.shape[0], mid, HW))

    kk = ksize * ksize
    core = w3f[:, None, :] * dwf.T[None, :, :]
    w3cat = jnp.pad(core, ((_MPAD, 0), (0, 0), (0, _CPAD - mid))) \
        .reshape(outputs + _MPAD, kk * _CPAD).astype(jnp.bfloat16)
    b3p = jnp.pad(b3f, (_MPAD, 0))[:, None]

    b1c = b1[:, None]
    Bb = _pick_block_batch(B)
    x3 = x.reshape(B, C, HW)

    params = (wcomb, b1c, pmask, w3cat, b3p)
    kern = functools.partial(_block_kernel, ksize=ksize, pad=pad, W=W, Bb=Bb,
                             inp=inp)
    out = pl.pallas_call(
        kern,
        out_shape=jax.ShapeDtypeStruct((B, C, HW), jnp.float32),
        grid=(B // Bb,),
        in_specs=[pl.BlockSpec((Bb, C, HW), lambda b: (b, 0, 0))]
                 + [_const_spec(a) for a in params],
        out_specs=pl.BlockSpec((Bb, C, HW), lambda b: (b, 0, 0)),
        compiler_params=pltpu.CompilerParams(
            dimension_semantics=("parallel",),
            vmem_limit_bytes=_VMEM_LIMIT),
    )(x3, *params)
    return out.reshape(B, C, H, W)


# speedup vs baseline: 1.0389x; 1.0097x over previous
import functools

import jax
import jax.numpy as jnp
from jax.experimental import pallas as pl
from jax.experimental.pallas import tpu as pltpu


_VMEM_LIMIT = 64 * 1024 * 1024
_BN_EPS = 1e-5
_CPAD = 128
_MPAD = 4


def _shift_lanes(a, d):
    if d == 0:
        return a
    C = a.shape[0]
    if d > 0:
        return jnp.concatenate(
            [a[:, d:], jnp.zeros((C, d), a.dtype)], axis=1)
    return jnp.concatenate(
        [jnp.zeros((C, -d), a.dtype), a[:, :a.shape[1] + d]], axis=1)


def _block_kernel(x_ref, wc_ref, b1_ref, pm_ref, w3cat_ref, b3p_ref,
                  o_ref, *, ksize, pad, W, Bb, inp):
    HW = x_ref.shape[-1]
    wc = wc_ref[...]
    b1 = b1_ref[...]
    w3cat = w3cat_ref[...]
    b3p = b3p_ref[...]

    zrows = jnp.zeros((_CPAD - inp, HW), jnp.bfloat16)
    for b in range(Bb):
        xb = x_ref[b].astype(jnp.bfloat16)
        r = jnp.dot(wc, xb, preferred_element_type=jnp.float32)
        proj = r[:inp]
        h = jnp.maximum(r[inp + _MPAD:] + b1, 0.0)
        hb0 = h.astype(jnp.bfloat16)
        hb = {ox: hb0 * pm_ref[i] for i, ox in
              enumerate(o for o in range(-pad, pad + 1) if o != 0)}
        hb[0] = hb0
        pieces = []
        for dy in range(ksize):
            oy = dy - pad
            for dx in range(ksize):
                ox = dx - pad
                pieces.append(_shift_lanes(hb[ox], oy * W + ox))
                pieces.append(zrows)
        stack = jnp.concatenate(pieces, axis=0)
        y = jnp.maximum(
            jnp.dot(w3cat, stack, preferred_element_type=jnp.float32) + b3p,
            0.0)
        o_ref[b, 2 * inp - y.shape[0]:] = y
        o_ref[b, :inp] = proj


def _const_spec(a):
    zeros = (0,) * a.ndim
    return pl.BlockSpec(a.shape, lambda b: zeros)


def _pick_block_batch(B, target_steps=16):
    cap = max(1, B // target_steps)
    for bb in range(cap, 0, -1):
        if B % bb == 0:
            return bb
    return 1


def _fold_bn(gamma, beta, mean, var, eps=_BN_EPS):
    scale = gamma / jnp.sqrt(var + eps)
    bias = beta - mean * scale
    return scale, bias


def kernel(x, main_w1, main_bn1_gamma, main_bn1_beta, main_bn1_mean,
           main_bn1_var, main_dw, main_bn2_gamma, main_bn2_beta, main_bn2_mean,
           main_bn2_var, main_w3, main_bn3_gamma, main_bn3_beta, main_bn3_mean,
           main_bn3_var):
    B, C, H, W = x.shape
    inp = C // 2
    mid = main_w1.shape[0]
    HW = H * W
    ksize = main_dw.shape[-1]
    pad = ksize // 2
    outputs = main_w3.shape[0]
    assert outputs == inp

    s1, b1 = _fold_bn(main_bn1_gamma, main_bn1_beta, main_bn1_mean, main_bn1_var)
    s2, b2 = _fold_bn(main_bn2_gamma, main_bn2_beta, main_bn2_mean, main_bn2_var)
    s3, b3 = _fold_bn(main_bn3_gamma, main_bn3_beta, main_bn3_mean, main_bn3_var)
    w1 = main_w1[:, :, 0, 0] * s1[:, None]
    dwf = main_dw[:, 0].reshape(mid, -1) * s2[:, None]
    w3f = main_w3[:, :, 0, 0] * s3[:, None]
    b3f = b3 + s3 * (main_w3[:, :, 0, 0] @ b2)

    w1e = jnp.stack([jnp.zeros_like(w1), w1], axis=-1) \
        .reshape(mid, C).astype(jnp.bfloat16)
    sel = jnp.stack([jnp.eye(inp, dtype=jnp.bfloat16),
                     jnp.zeros((inp, inp), jnp.bfloat16)], axis=-1) \
        .reshape(inp, C)
    wcomb = jnp.concatenate(
        [sel, jnp.zeros((_MPAD, C), jnp.bfloat16), w1e], axis=0)

    xpos = jnp.arange(W)
    pmrows = [jnp.tile(((xpos >= max(0, ox)) & (xpos < W + min(0, ox))), H)
              for ox in range(-pad, pad + 1) if ox != 0]
    pmask = jnp.stack(pmrows).astype(jnp.bfloat16)
    pmask = jnp.broadcast_to(pmask[:, None, :], (pmask.shape[0], mid, HW))

    kk = ksize * ksize
    core = w3f[:, None, :] * dwf.T[None, :, :]
    w3cat = jnp.pad(core, ((_MPAD, 0), (0, 0), (0, _CPAD - mid))) \
        .reshape(outputs + _MPAD, kk * _CPAD).astype(jnp.bfloat16)
    b3p = jnp.pad(b3f, (_MPAD, 0))[:, None]

    b1c = b1[:, None]
    Bb = _pick_block_batch(B)
    x3 = x.reshape(B, C, HW)

    params = (wcomb, b1c, pmask, w3cat, b3p)
    kern = functools.partial(_block_kernel, ksize=ksize, pad=pad, W=W, Bb=Bb,
                             inp=inp)
    out = pl.pallas_call(
        kern,
        out_shape=jax.ShapeDtypeStruct((B, C, HW), jnp.float32),
        grid=(B // Bb,),
        in_specs=[pl.BlockSpec((Bb, C, HW), lambda b: (b, 0, 0))]
                 + [_const_spec(a) for a in params],
        out_specs=pl.BlockSpec((Bb, C, HW), lambda b: (b, 0, 0)),
        compiler_params=pltpu.CompilerParams(
            dimension_semantics=("parallel",),
            vmem_limit_bytes=_VMEM_LIMIT),
    )(x3, *params)
    return out.reshape(B, C, H, W)
